# Initial kernel scaffold; baseline (speedup 1.0000x reference)
import jax
import jax.numpy as jnp
from jax.experimental import pallas as pl


def kernel(indices, table):
    raise NotImplementedError("write your pallas kernel here")


# speedup vs baseline: 4.1373x; 4.1373x over previous
import functools

import jax
import jax.numpy as jnp
from jax import lax
from jax.experimental import pallas as pl
from jax.experimental.pallas import tpu as pltpu
from jax.experimental.pallas import tpu_sc as plsc

BATCH = 4096
SEQ_LEN = 200
EMBED_DIM = 64
TOTAL = BATCH * SEQ_LEN

_INFO = plsc.get_sparse_core_info()
NC = _INFO.num_cores
NS = _INFO.num_subcores
NW = NC * NS

ROWS_PER_STREAM = 128
ROWS_PER_WORKER = TOTAL // NW
STREAMS_PER_CHUNK = 8
CHUNK_ROWS = STREAMS_PER_CHUNK * ROWS_PER_STREAM
CHUNKS_PER_WORKER = ROWS_PER_WORKER // CHUNK_ROWS
STREAMS_PER_WORKER = ROWS_PER_WORKER // ROWS_PER_STREAM


def _gather_body(idx_hbm, table_hbm, out_hbm, idx_v, rows_v, sem):
    wid = lax.axis_index("s") * NC + lax.axis_index("c")
    stream0 = wid * STREAMS_PER_WORKER
    row0 = wid * ROWS_PER_WORKER

    def chunk(c, carry):
        sbase = stream0 + c * STREAMS_PER_CHUNK
        rbase = row0 + c * CHUNK_ROWS
        pltpu.sync_copy(idx_hbm.at[pl.ds(sbase, STREAMS_PER_CHUNK), :], idx_v)
        handles = [
            pltpu.async_copy(
                table_hbm.at[idx_v.at[j]],
                rows_v.at[pl.ds(j * ROWS_PER_STREAM, ROWS_PER_STREAM), :],
                sem,
            )
            for j in range(STREAMS_PER_CHUNK)
        ]
        for h in handles:
            h.wait()
        pltpu.sync_copy(rows_v, out_hbm.at[pl.ds(rbase, CHUNK_ROWS), :])
        return carry

    lax.fori_loop(0, CHUNKS_PER_WORKER, chunk, 0)


@jax.jit
def _sc_gather(idx2d, table):
    mesh = plsc.VectorSubcoreMesh(core_axis_name="c", subcore_axis_name="s")
    run = functools.partial(
        pl.kernel,
        out_type=jax.ShapeDtypeStruct((TOTAL, EMBED_DIM), jnp.float32),
        mesh=mesh,
        compiler_params=pltpu.CompilerParams(use_tc_tiling_on_sc=False),
        scratch_types=[
            pltpu.VMEM((STREAMS_PER_CHUNK, ROWS_PER_STREAM), jnp.int32),
            pltpu.VMEM((CHUNK_ROWS, EMBED_DIM), jnp.float32),
            pltpu.SemaphoreType.DMA,
        ],
    )(_gather_body)
    return run(idx2d, table)


def kernel(indices, table):
    idx2d = indices.astype(jnp.int32).reshape(TOTAL // ROWS_PER_STREAM,
                                              ROWS_PER_STREAM)
    out = _sc_gather(idx2d, table)
    return out.reshape(BATCH, SEQ_LEN, EMBED_DIM)


# speedup vs baseline: 4.2656x; 1.0310x over previous
import functools

import jax
import jax.numpy as jnp
from jax import lax
from jax.experimental import pallas as pl
from jax.experimental.pallas import tpu as pltpu
from jax.experimental.pallas import tpu_sc as plsc

BATCH = 4096
SEQ_LEN = 200
EMBED_DIM = 64
TOTAL = BATCH * SEQ_LEN

_INFO = plsc.get_sparse_core_info()
NC = _INFO.num_cores
NS = _INFO.num_subcores
NW = NC * NS

ROWS_PER_STREAM = 128
ROWS_PER_WORKER = TOTAL // NW
STREAMS_PER_WORKER = ROWS_PER_WORKER // ROWS_PER_STREAM
STREAMS_PER_CHUNK = 2
CHUNK_ROWS = STREAMS_PER_CHUNK * ROWS_PER_STREAM
CHUNKS_PER_WORKER = ROWS_PER_WORKER // CHUNK_ROWS
NBUF = 4


def _gather_body(idx_hbm, table_hbm, out_hbm, idx_all, *scratch):
    rows = scratch[:NBUF]
    gsem = scratch[NBUF:2 * NBUF]
    wsem = scratch[2 * NBUF:3 * NBUF]

    wid = lax.axis_index("s") * NC + lax.axis_index("c")
    stream0 = wid * STREAMS_PER_WORKER
    row0 = wid * ROWS_PER_WORKER

    pltpu.sync_copy(idx_hbm.at[pl.ds(stream0, STREAMS_PER_WORKER), :], idx_all)

    def fire_gathers(c, b):
        for j in range(STREAMS_PER_CHUNK):
            pltpu.async_copy(
                table_hbm.at[idx_all.at[c * STREAMS_PER_CHUNK + j]],
                rows[b].at[pl.ds(j * ROWS_PER_STREAM, ROWS_PER_STREAM), :],
                gsem[b],
            )

    def drain(sem, b):
        pltpu.make_async_copy(
            out_hbm.at[pl.ds(0, CHUNK_ROWS), :], rows[b], sem).wait()

    def writeback(c, b):
        pltpu.async_copy(
            rows[b], out_hbm.at[pl.ds(row0 + c * CHUNK_ROWS, CHUNK_ROWS), :],
            wsem[b])

    for b in range(NBUF):
        fire_gathers(b, b)

    main_slots = CHUNKS_PER_WORKER - NBUF

    @pl.loop(0, main_slots, step=NBUF)
    def _(c0):
        for b in range(NBUF):
            c = c0 + b
            drain(gsem[b], b)
            writeback(c, b)
            drain(wsem[b], b)
            fire_gathers(c + NBUF, b)

    for b in range(NBUF):
        c = main_slots + b
        drain(gsem[b], b)
        writeback(c, b)
        drain(wsem[b], b)


@jax.jit
def _sc_gather(idx2d, table):
    mesh = plsc.VectorSubcoreMesh(core_axis_name="c", subcore_axis_name="s")
    run = functools.partial(
        pl.kernel,
        out_type=jax.ShapeDtypeStruct((TOTAL, EMBED_DIM), jnp.float32),
        mesh=mesh,
        compiler_params=pltpu.CompilerParams(use_tc_tiling_on_sc=False),
        scratch_types=[
            pltpu.VMEM((STREAMS_PER_WORKER, ROWS_PER_STREAM), jnp.int32),
            *[pltpu.VMEM((CHUNK_ROWS, EMBED_DIM), jnp.float32)
              for _ in range(NBUF)],
            *[pltpu.SemaphoreType.DMA for _ in range(2 * NBUF)],
        ],
    )(_gather_body)
    return run(idx2d, table)


def kernel(indices, table):
    idx2d = indices.astype(jnp.int32).reshape(TOTAL // ROWS_PER_STREAM,
                                              ROWS_PER_STREAM)
    out = _sc_gather(idx2d, table)
    return out.reshape(BATCH, SEQ_LEN, EMBED_DIM)
